# baseline (device time: 9042 ns/iter reference)
import jax
import jax.numpy as jnp
from jax import lax
from jax.experimental import pallas as pl
from jax.experimental.pallas import tpu as pltpu

N_DEV = 8
PLANE = 4


def kernel(x):
    m_per, n = x.shape

    def body(x_hbm, out_hbm, x_vmem, out_vmem, comm_ref, send_sems, recv_sems,
             ready_sems, in_sem, out_sem):
        my_pos = lax.axis_index("i")
        mirror = lax.rem(my_pos + PLANE, N_DEV)
        j = lax.rem(my_pos, PLANE)
        base = my_pos - j

        def plane_peer(t):
            return base + lax.rem(j + t, PLANE)

        barrier_sem = pltpu.get_barrier_semaphore()
        pl.semaphore_signal(
            barrier_sem, inc=1,
            device_id=(mirror,), device_id_type=pl.DeviceIdType.MESH,
        )
        for t in range(1, PLANE):
            pl.semaphore_signal(
                ready_sems.at[PLANE - t], inc=1,
                device_id=(plane_peer(t),), device_id_type=pl.DeviceIdType.MESH,
            )

        in_copy = pltpu.make_async_copy(x_hbm, x_vmem, in_sem)
        in_copy.start()
        in_copy.wait()
        comm_ref[0, :, :] = jnp.sum(x_vmem[:, :], axis=0, keepdims=True)

        pl.semaphore_wait(barrier_sem, 1)
        rdma_a = pltpu.make_async_remote_copy(
            src_ref=comm_ref.at[0],
            dst_ref=comm_ref.at[1],
            send_sem=send_sems.at[0],
            recv_sem=recv_sems.at[1],
            device_id=(mirror,),
            device_id_type=pl.DeviceIdType.MESH,
        )
        rdma_a.start()
        rdma_a.wait()
        comm_ref[2, :, :] = comm_ref[0, :, :] + comm_ref[1, :, :]

        rdmas = []
        for t in (2, 1, 3):
            pl.semaphore_wait(ready_sems.at[t], 1)
            rdma = pltpu.make_async_remote_copy(
                src_ref=comm_ref.at[2],
                dst_ref=comm_ref.at[2 + t],
                send_sem=send_sems.at[2 + t],
                recv_sem=recv_sems.at[2 + t],
                device_id=(plane_peer(t),),
                device_id_type=pl.DeviceIdType.MESH,
            )
            rdma.start()
            rdmas.append(rdma)
        for rdma in rdmas:
            rdma.wait()

        out_vmem[:, :] = jnp.sum(comm_ref[2:6, 0, :], axis=0, keepdims=True)
        out_copy = pltpu.make_async_copy(out_vmem, out_hbm, out_sem)
        out_copy.start()
        out_copy.wait()

    return pl.pallas_call(
        body,
        out_shape=jax.ShapeDtypeStruct((1, n), jnp.float32),
        in_specs=[pl.BlockSpec(memory_space=pl.ANY)],
        out_specs=pl.BlockSpec(memory_space=pl.ANY),
        scratch_shapes=[
            pltpu.VMEM((m_per, n), jnp.float32),
            pltpu.VMEM((1, n), jnp.float32),
            pltpu.VMEM((6, 1, n), jnp.float32),
            pltpu.SemaphoreType.DMA((6,)),
            pltpu.SemaphoreType.DMA((6,)),
            pltpu.SemaphoreType.REGULAR((PLANE,)),
            pltpu.SemaphoreType.DMA,
            pltpu.SemaphoreType.DMA,
        ],
        compiler_params=pltpu.CompilerParams(collective_id=0),
    )(x)


# device time: 8034 ns/iter; 1.1255x vs baseline; 1.1255x over previous
import jax
import jax.numpy as jnp
from jax import lax
from jax.experimental import pallas as pl
from jax.experimental.pallas import tpu as pltpu

N_DEV = 8


def kernel(x):
    m_per, n = x.shape

    def body(x_ref, out_ref, comm_ref, send_sems, recv_sems):
        my_pos = lax.axis_index("i")

        barrier_sem = pltpu.get_barrier_semaphore()
        for k in range(1, N_DEV):
            pl.semaphore_signal(
                barrier_sem,
                inc=1,
                device_id=((my_pos + k) % N_DEV,),
                device_id_type=pl.DeviceIdType.MESH,
            )

        comm_ref[0, :, :] = jnp.sum(x_ref[:, :], axis=0, keepdims=True)

        pl.semaphore_wait(barrier_sem, N_DEV - 1)

        rdmas = []
        for k in range(1, N_DEV):
            rdma = pltpu.make_async_remote_copy(
                src_ref=comm_ref.at[0],
                dst_ref=comm_ref.at[k],
                send_sem=send_sems.at[k],
                recv_sem=recv_sems.at[k],
                device_id=((my_pos + k) % N_DEV,),
                device_id_type=pl.DeviceIdType.MESH,
            )
            rdma.start()
            rdmas.append(rdma)

        for rdma in rdmas:
            rdma.wait()

        out_ref[:, :] = jnp.sum(comm_ref[:, 0, :], axis=0, keepdims=True)

    return pl.pallas_call(
        body,
        out_shape=jax.ShapeDtypeStruct((1, n), jnp.float32),
        in_specs=[pl.BlockSpec(memory_space=pltpu.VMEM)],
        out_specs=pl.BlockSpec(memory_space=pltpu.VMEM),
        scratch_shapes=[
            pltpu.VMEM((N_DEV, 1, n), jnp.float32),
            pltpu.SemaphoreType.DMA((N_DEV,)),
            pltpu.SemaphoreType.DMA((N_DEV,)),
        ],
        compiler_params=pltpu.CompilerParams(collective_id=0),
    )(x)
